# Initial kernel scaffold; baseline (speedup 1.0000x reference)
import jax
import jax.numpy as jnp
from jax.experimental import pallas as pl


def kernel(features, tokens, pos_table):
    raise NotImplementedError("write your pallas kernel here")


# speedup vs baseline: 1.4647x; 1.4647x over previous
import jax
import jax.numpy as jnp
from jax.experimental import pallas as pl

SEQ_LEN = 2048
OUT_DIM = 1024
BATCH = 4
L_BLOCK = 256


def _add_body(f_ref, p_ref, o_ref):
    o_ref[...] = f_ref[...] + p_ref[...]


def kernel(features, tokens, pos_table):
    del tokens
    B, L, D = features.shape
    grid = (L // L_BLOCK, B)
    return pl.pallas_call(
        _add_body,
        grid=grid,
        in_specs=[
            pl.BlockSpec((1, L_BLOCK, D), lambda i, j: (j, i, 0)),
            pl.BlockSpec((L_BLOCK, D), lambda i, j: (i, 0)),
        ],
        out_specs=pl.BlockSpec((1, L_BLOCK, D), lambda i, j: (j, i, 0)),
        out_shape=jax.ShapeDtypeStruct((B, L, D), features.dtype),
    )(features, pos_table)


# speedup vs baseline: 2.1667x; 1.4793x over previous
import jax
import jax.numpy as jnp
from jax.experimental import pallas as pl

SEQ_LEN = 2048
OUT_DIM = 1024
BATCH = 4
L_BLOCK = 256


def _add_body(f_ref, p_ref, o_ref):
    o_ref[...] = f_ref[...] + p_ref[...]


def kernel(features, tokens, pos_table):
    del tokens
    B, L, D = features.shape
    grid = (L // L_BLOCK,)
    return pl.pallas_call(
        _add_body,
        grid=grid,
        in_specs=[
            pl.BlockSpec((B, L_BLOCK, D), lambda i: (0, i, 0)),
            pl.BlockSpec((L_BLOCK, D), lambda i: (i, 0)),
        ],
        out_specs=pl.BlockSpec((B, L_BLOCK, D), lambda i: (0, i, 0)),
        out_shape=jax.ShapeDtypeStruct((B, L, D), features.dtype),
    )(features, pos_table)


# speedup vs baseline: 2.1676x; 1.0004x over previous
import jax
import jax.numpy as jnp
from jax.experimental import pallas as pl

SEQ_LEN = 2048
OUT_DIM = 1024
BATCH = 4
L_BLOCK = 512


def _add_body(f_ref, p_ref, o_ref):
    o_ref[...] = f_ref[...] + p_ref[...]


def kernel(features, tokens, pos_table):
    del tokens
    B, L, D = features.shape
    grid = (L // L_BLOCK,)
    return pl.pallas_call(
        _add_body,
        grid=grid,
        in_specs=[
            pl.BlockSpec((B, L_BLOCK, D), lambda i: (0, i, 0)),
            pl.BlockSpec((L_BLOCK, D), lambda i: (i, 0)),
        ],
        out_specs=pl.BlockSpec((B, L_BLOCK, D), lambda i: (0, i, 0)),
        out_shape=jax.ShapeDtypeStruct((B, L, D), features.dtype),
    )(features, pos_table)
